# baseline (speedup 1.0000x reference)
import functools

import jax
import jax.numpy as jnp
from jax import lax
from jax.experimental import pallas as pl
from jax.experimental.pallas import tpu as pltpu
from jax.experimental.pallas import tpu_sc as plsc

_B = 1024
_WALK = 50
_NEG = 20
_D = 16
_NTOT = _B * (1 + _WALK + _NEG)
_NW = 32
_PER_W = 2304
_NPAD = _NW * _PER_W
_CHUNK = 128
_NCHUNK = _PER_W // _CHUNK


def _sc_gather(table, idx):
    mesh = plsc.VectorSubcoreMesh(core_axis_name="c", subcore_axis_name="s")

    @functools.partial(
        pl.kernel,
        mesh=mesh,
        out_type=jax.ShapeDtypeStruct((_NPAD, _D), jnp.float32),
        scratch_types=[
            pltpu.VMEM((_PER_W,), jnp.int32),
            pltpu.VMEM((_PER_W, _D), jnp.float32),
            pltpu.SemaphoreType.DMA,
        ],
        compiler_params=pltpu.CompilerParams(use_tc_tiling_on_sc=False),
    )
    def k(table_hbm, idx_hbm, out_hbm, idx_v, rows_v, sem):
        nc = 2
        wid = lax.axis_index("s") * nc + lax.axis_index("c")
        base = wid * _PER_W
        pltpu.sync_copy(idx_hbm.at[pl.ds(base, _PER_W)], idx_v)
        copies = []
        for c in range(_NCHUNK):
            copies.append(
                pltpu.async_copy(
                    table_hbm.at[idx_v.at[pl.ds(c * _CHUNK, _CHUNK)]],
                    rows_v.at[pl.ds(c * _CHUNK, _CHUNK)],
                    sem,
                )
            )
        for cp in copies:
            cp.wait()
        pltpu.sync_copy(rows_v, out_hbm.at[pl.ds(base, _PER_W)])

    return k(table, idx)


def _dot(a, b):
    return jnp.dot(a, b, precision=lax.Precision.HIGHEST,
                   preferred_element_type=jnp.float32)


def _clip_scale(ss):
    return jnp.minimum(1.0, lax.rsqrt(jnp.maximum(ss, 1e-24)))


def _stats_body(nodes_ref, walks_ref, negs_ref, s50_ref, s50t_ref, u50_ref,
                s20_ref, s20t_ref, u20_ref,
                net_ref, loss_ref):
    e = nodes_ref[...]
    ss = jnp.sum(e * e, axis=1, keepdims=True)
    ne = e * _clip_scale(ss)
    net_ref[...] = jnp.transpose(ne)

    w = walks_ref[...]
    ssw = _dot(w * w, s50_ref[...])
    cw = w * _dot(_clip_scale(ssw), s50t_ref[...])
    net50 = _dot(ne, u50_ref[...])
    wsum = jnp.sum(cw * net50, axis=1, keepdims=True)

    g = negs_ref[...]
    ssn = _dot(g * g, s20_ref[...])
    cg = g * _dot(_clip_scale(ssn), s20t_ref[...])
    net20 = _dot(ne, u20_ref[...])
    sim = _dot(cg * net20, s20_ref[...])
    nsum = jnp.sum(jnp.exp(sim), axis=1, keepdims=True)
    loss_ref[0, 0] = jnp.sum(jnp.log(nsum) - wsum)


def _edge_body(nei_ref, net_ref, out_ref):
    out_ref[...] = nei_ref[...] * net_ref[...][None, :, :]


def _seg_onehot(width, d):
    r = lax.broadcasted_iota(jnp.int32, (width * d, width), 0) // d
    c = lax.broadcasted_iota(jnp.int32, (width * d, width), 1)
    return (r == c).astype(jnp.float32)


def _tile_onehot(n_lanes, d):
    r = lax.broadcasted_iota(jnp.int32, (d, n_lanes), 0)
    c = lax.broadcasted_iota(jnp.int32, (d, n_lanes), 1) % d
    return (r == c).astype(jnp.float32)


def kernel(nodes, walks, neg_samples, node_embedding_var):
    idx_all = jnp.concatenate(
        [nodes, walks.reshape(-1), neg_samples.reshape(-1),
         jnp.zeros((_NPAD - _NTOT,), jnp.int32)]
    )
    gathered = _sc_gather(node_embedding_var, idx_all)
    nodes_g = gathered[:_B]
    walks_v = gathered[_B:_B * (1 + _WALK)].reshape(_B, _WALK * _D)
    negs_v = gathered[_B * (1 + _WALK):_NTOT].reshape(_B, _NEG * _D)

    s50 = _seg_onehot(_WALK, _D)
    s50t = s50.T
    u50 = _tile_onehot(_WALK * _D, _D)
    s20 = _seg_onehot(_NEG, _D)
    s20t = s20.T
    u20 = _tile_onehot(_NEG * _D, _D)

    net, loss = pl.pallas_call(
        _stats_body,
        out_shape=(
            jax.ShapeDtypeStruct((_D, _B), jnp.float32),
            jax.ShapeDtypeStruct((1, 1), jnp.float32),
        ),
        out_specs=(
            pl.BlockSpec(memory_space=pltpu.VMEM),
            pl.BlockSpec(memory_space=pltpu.SMEM),
        ),
    )(nodes_g, walks_v, negs_v, s50, s50t, u50, s20, s20t, u20)

    ne = jnp.transpose(net)
    nei = ne.reshape(_B, _D, 1)
    edge_t = pl.pallas_call(
        _edge_body,
        grid=(16,),
        in_specs=[
            pl.BlockSpec((64, _D, 1), lambda i: (i, 0, 0)),
            pl.BlockSpec((_D, _B), lambda i: (0, 0)),
        ],
        out_specs=pl.BlockSpec((64, _D, _B), lambda i: (i, 0, 0)),
        out_shape=jax.ShapeDtypeStruct((_B, _D, _B), jnp.float32),
    )(nei, net)
    edge = jnp.transpose(edge_t, (0, 2, 1))
    return loss[0, 0], ne, edge
